# Initial kernel scaffold; baseline (speedup 1.0000x reference)
import jax
import jax.numpy as jnp
from jax.experimental import pallas as pl


def kernel(inputs, emb_weight):
    raise NotImplementedError("write your pallas kernel here")


# speedup vs baseline: 1.9991x; 1.9991x over previous
import functools

import jax
import jax.numpy as jnp
from jax import lax
from jax.experimental import pallas as pl
from jax.experimental.pallas import tpu as pltpu
from jax.experimental.pallas import tpu_sc as plsc

B = 1024
T = 200
V = 1000
D = 128
L = 16

NC = 2
NS = 16
NW = NC * NS

DGROUPS = 2
DH = D // DGROUPS
BGROUPS = NW // DGROUPS
BPW = B // BGROUPS

_T_FULL = T // L
_T_TAIL = T - L


def _body(tab_hbm, idx_hbm, out_hbm, tab_v, idx_v, stage_v, sem0, sem1):
    cid = lax.axis_index("c")
    sid = lax.axis_index("s")
    wid = sid * NC + cid
    dgi = wid % DGROUPS
    bgi = wid // DGROUPS
    d0 = dgi * DH
    b0 = bgi * BPW

    pltpu.sync_copy(tab_hbm.at[pl.ds(d0 * V, DH * V)], tab_v)
    pltpu.sync_copy(idx_hbm.at[pl.ds(b0, BPW)], idx_v)

    sems = (sem0, sem1)

    def compute_batch(b_local, buf):
        def chunk(t0):
            base = idx_v[b_local, pl.ds(t0, L)]
            for d in range(DH):
                vals = plsc.load_gather(tab_v, [base + d * V])
                stage_v[buf, d, pl.ds(t0, L)] = vals

        def tc_body(tc, carry):
            chunk(pl.multiple_of(tc * L, L))
            return carry

        lax.fori_loop(0, _T_FULL, tc_body, 0, unroll=False)
        chunk(_T_TAIL)

    def pair_body(i, carry):
        for k in range(2):
            b_local = i * 2 + k
            @pl.when(i > 0)
            def _wait():
                pltpu.make_async_copy(
                    stage_v.at[k],
                    out_hbm.at[0, pl.ds(0, DH)],
                    sems[k],
                ).wait()

            compute_batch(b_local, k)
            pltpu.async_copy(
                stage_v.at[k],
                out_hbm.at[b0 + b_local, pl.ds(d0, DH)],
                sems[k],
            )
        return carry

    lax.fori_loop(0, BPW // 2, pair_body, 0, unroll=False)

    for k in range(2):
        pltpu.make_async_copy(
            stage_v.at[k], out_hbm.at[0, pl.ds(0, DH)], sems[k]
        ).wait()


@jax.jit
def _lookup_transpose(tab_flat, idx):
    mesh = plsc.VectorSubcoreMesh(
        core_axis_name="c", subcore_axis_name="s", num_cores=NC, num_subcores=NS
    )
    return pl.kernel(
        _body,
        out_type=jax.ShapeDtypeStruct((B, D, T), jnp.float32),
        mesh=mesh,
        compiler_params=pltpu.CompilerParams(needs_layout_passes=False),
        scratch_types=[
            pltpu.VMEM((DH * V,), jnp.float32),
            pltpu.VMEM((BPW, T), jnp.int32),
            pltpu.VMEM((2, DH, T), jnp.float32),
            pltpu.SemaphoreType.DMA,
            pltpu.SemaphoreType.DMA,
        ],
    )(tab_flat, idx)


def kernel(inputs, emb_weight):
    tab_flat = jnp.transpose(emb_weight).reshape(-1)
    idx = inputs.astype(jnp.int32)
    return _lookup_transpose(tab_flat, idx)


# speedup vs baseline: 3.6655x; 1.8336x over previous
import functools

import jax
import jax.numpy as jnp
from jax import lax
from jax.experimental import pallas as pl
from jax.experimental.pallas import tpu as pltpu
from jax.experimental.pallas import tpu_sc as plsc

B = 1024
T = 200
V = 1000
D = 128
L = 16

NC = 2
NS = 16
NW = NC * NS

DGROUPS = 2
DH = D // DGROUPS
BGROUPS = NW // DGROUPS
BPW = B // BGROUPS

_T_FULL = T // L
_T_TAIL = T - L


def _body(tab_hbm, idx_hbm, out_hbm, tab_v, idx_v, stage_v, sem0, sem1):
    cid = lax.axis_index("c")
    sid = lax.axis_index("s")
    wid = sid * NC + cid
    dgi = wid % DGROUPS
    bgi = wid // DGROUPS
    d0 = dgi * DH
    b0 = bgi * BPW

    pltpu.sync_copy(tab_hbm.at[pl.ds(d0 * V, DH * V)], tab_v)
    pltpu.sync_copy(idx_hbm.at[pl.ds(b0, BPW)], idx_v)

    sems = (sem0, sem1)

    def compute_batch(b_local, buf):
        def chunk(t0):
            base = idx_v[b_local, pl.ds(t0, L)]
            G = 8
            for g0 in range(0, DH, G):
                vals = [
                    plsc.load_gather(tab_v, [base + (g0 + j) * V])
                    for j in range(G)
                ]
                for j in range(G):
                    stage_v[buf, g0 + j, pl.ds(t0, L)] = vals[j]

        def tc_body(tc, carry):
            chunk(pl.multiple_of(tc * L, L))
            return carry

        lax.fori_loop(0, _T_FULL, tc_body, 0, unroll=False)
        chunk(_T_TAIL)

    def pair_body(i, carry):
        for k in range(2):
            b_local = i * 2 + k
            @pl.when(i > 0)
            def _wait():
                pltpu.make_async_copy(
                    stage_v.at[k],
                    out_hbm.at[0, pl.ds(0, DH)],
                    sems[k],
                ).wait()

            compute_batch(b_local, k)
            pltpu.async_copy(
                stage_v.at[k],
                out_hbm.at[b0 + b_local, pl.ds(d0, DH)],
                sems[k],
            )
        return carry

    lax.fori_loop(0, BPW // 2, pair_body, 0, unroll=False)

    for k in range(2):
        pltpu.make_async_copy(
            stage_v.at[k], out_hbm.at[0, pl.ds(0, DH)], sems[k]
        ).wait()


@jax.jit
def _lookup_transpose(tab_flat, idx):
    mesh = plsc.VectorSubcoreMesh(
        core_axis_name="c", subcore_axis_name="s", num_cores=NC, num_subcores=NS
    )
    return pl.kernel(
        _body,
        out_type=jax.ShapeDtypeStruct((B, D, T), jnp.float32),
        mesh=mesh,
        compiler_params=pltpu.CompilerParams(needs_layout_passes=False),
        scratch_types=[
            pltpu.VMEM((DH * V,), jnp.float32),
            pltpu.VMEM((BPW, T), jnp.int32),
            pltpu.VMEM((2, DH, T), jnp.float32),
            pltpu.SemaphoreType.DMA,
            pltpu.SemaphoreType.DMA,
        ],
    )(tab_flat, idx)


def kernel(inputs, emb_weight):
    tab_flat = jnp.transpose(emb_weight).reshape(-1)
    idx = inputs.astype(jnp.int32)
    return _lookup_transpose(tab_flat, idx)


# speedup vs baseline: 3.7388x; 1.0200x over previous
import functools

import jax
import jax.numpy as jnp
from jax import lax
from jax.experimental import pallas as pl
from jax.experimental.pallas import tpu as pltpu
from jax.experimental.pallas import tpu_sc as plsc

B = 1024
T = 200
V = 1000
D = 128
L = 16

NC = 2
NS = 16
NW = NC * NS

DGROUPS = 2
DH = D // DGROUPS
BGROUPS = NW // DGROUPS
BPW = B // BGROUPS

_T_FULL = T // L
_T_TAIL = T - L


def _body(tab_hbm, idx_hbm, out_hbm, tab_v, idx_v, stage_v, sem0, sem1):
    cid = lax.axis_index("c")
    sid = lax.axis_index("s")
    wid = sid * NC + cid
    dgi = wid % DGROUPS
    bgi = wid // DGROUPS
    d0 = dgi * DH
    b0 = bgi * BPW

    pltpu.sync_copy(tab_hbm.at[pl.ds(d0 * V, DH * V)], tab_v)
    pltpu.sync_copy(idx_hbm.at[pl.ds(b0, BPW)], idx_v)

    sems = (sem0, sem1)

    def compute_batch(b_local, buf):
        def chunk(t0):
            base = idx_v[b_local, pl.ds(t0, L)]
            G = 8
            def loads(g0):
                return [
                    plsc.load_gather(tab_v, [base + (g0 + j) * V])
                    for j in range(G)
                ]

            prev = loads(0)
            for g0 in range(G, DH, G):
                cur = []
                for j in range(G):
                    cur.append(plsc.load_gather(tab_v, [base + (g0 + j) * V]))
                    stage_v[buf, g0 - G + j, pl.ds(t0, L)] = prev[j]
                prev = cur
            for j in range(G):
                stage_v[buf, DH - G + j, pl.ds(t0, L)] = prev[j]

        def tc_body(tc, carry):
            chunk(pl.multiple_of(tc * L, L))
            return carry

        lax.fori_loop(0, _T_FULL, tc_body, 0, unroll=False)
        chunk(_T_TAIL)

    def pair_body(i, carry):
        for k in range(2):
            b_local = i * 2 + k
            @pl.when(i > 0)
            def _wait():
                pltpu.make_async_copy(
                    stage_v.at[k],
                    out_hbm.at[0, pl.ds(0, DH)],
                    sems[k],
                ).wait()

            compute_batch(b_local, k)
            pltpu.async_copy(
                stage_v.at[k],
                out_hbm.at[b0 + b_local, pl.ds(d0, DH)],
                sems[k],
            )
        return carry

    lax.fori_loop(0, BPW // 2, pair_body, 0, unroll=False)

    for k in range(2):
        pltpu.make_async_copy(
            stage_v.at[k], out_hbm.at[0, pl.ds(0, DH)], sems[k]
        ).wait()


@jax.jit
def _lookup_transpose(tab_flat, idx):
    mesh = plsc.VectorSubcoreMesh(
        core_axis_name="c", subcore_axis_name="s", num_cores=NC, num_subcores=NS
    )
    return pl.kernel(
        _body,
        out_type=jax.ShapeDtypeStruct((B, D, T), jnp.float32),
        mesh=mesh,
        compiler_params=pltpu.CompilerParams(needs_layout_passes=False),
        scratch_types=[
            pltpu.VMEM((DH * V,), jnp.float32),
            pltpu.VMEM((BPW, T), jnp.int32),
            pltpu.VMEM((2, DH, T), jnp.float32),
            pltpu.SemaphoreType.DMA,
            pltpu.SemaphoreType.DMA,
        ],
    )(tab_flat, idx)


def kernel(inputs, emb_weight):
    tab_flat = jnp.transpose(emb_weight).reshape(-1)
    idx = inputs.astype(jnp.int32)
    return _lookup_transpose(tab_flat, idx)
